# baseline (speedup 1.0000x reference)
import functools

import jax
import jax.numpy as jnp
from jax import lax
from jax.experimental import pallas as pl
from jax.experimental.pallas import tpu as pltpu
from jax.experimental.pallas import tpu_sc as plsc

B = 1024
L = 200
F = 16
V = 100000

LANES = 16
LPAD = 208
NVEC = LPAD // LANES
PAD_ID = V
TABLE_W = 100096
CW = 4 * LPAD
BBLK = 256


def _sc_kernel(rows_per, row0, src_hbm, dst_hbm, cnt_hbm,
               table, ids_sA, ids_dA, ids_sB, ids_dB, cntA, cntB,
               sem_isA, sem_idA, sem_isB, sem_idB, sem_cA, sem_cB):
    wid = lax.axis_index("s") * 2 + lax.axis_index("c")
    base_row = wid * rows_per
    n_pairs = rows_per // 2

    zeros16 = jnp.zeros((LANES,), jnp.int32)
    ones16 = jnp.ones((LANES,), jnp.int32)

    def zinit(i, carry):
        for u in range(8):
            table[pl.ds((i * 8 + u) * LANES, LANES)] = zeros16
        return carry
    lax.fori_loop(0, TABLE_W // (8 * LANES), zinit, 0)

    pltpu.async_copy(src_hbm.at[row0 + base_row], ids_sA, sem_isA)
    pltpu.async_copy(dst_hbm.at[row0 + base_row], ids_dA, sem_idA)
    pltpu.async_copy(src_hbm.at[row0 + base_row + 1], ids_sB, sem_isB)
    pltpu.async_copy(dst_hbm.at[row0 + base_row + 1], ids_dB, sem_idB)

    def process(k, row, ids_s, ids_d, sem_is, sem_id, cnt, sem_c):
        pltpu.make_async_copy(src_hbm.at[row0 + row], ids_s, sem_is).wait()
        pltpu.make_async_copy(dst_hbm.at[row0 + row], ids_d, sem_id).wait()

        sv = [ids_s[pl.ds(v * LANES, LANES)] for v in range(NVEC)]
        dv = [ids_d[pl.ds(v * LANES, LANES)] for v in range(NVEC)]

        @pl.when(k + 1 < n_pairs)
        def _():
            pltpu.async_copy(src_hbm.at[row0 + row + 2], ids_s, sem_is)
            pltpu.async_copy(dst_hbm.at[row0 + row + 2], ids_d, sem_id)

        @pl.when(k > 0)
        def _():
            pltpu.make_async_copy(cnt, cnt_hbm.at[row - 2], sem_c).wait()

        for v in range(NVEC):
            plsc.addupdate_scatter(table, [sv[v]], ones16)
        for v in range(NVEC):
            cs = plsc.load_gather(table, [sv[v]])
            cd = plsc.load_gather(table, [dv[v]])
            cnt[pl.ds(v * LANES, LANES)] = jnp.where(sv[v] == 0, 0, cs)
            cnt[pl.ds(3 * LPAD + v * LANES, LANES)] = jnp.where(dv[v] == 0, 0, cd)
        for v in range(NVEC):
            plsc.store_scatter(table, [sv[v]], zeros16)

        for v in range(NVEC):
            plsc.addupdate_scatter(table, [dv[v]], ones16)
        for v in range(NVEC):
            cd = plsc.load_gather(table, [dv[v]])
            cs = plsc.load_gather(table, [sv[v]])
            cnt[pl.ds(2 * LPAD + v * LANES, LANES)] = jnp.where(dv[v] == 0, 0, cd)
            cnt[pl.ds(LPAD + v * LANES, LANES)] = jnp.where(sv[v] == 0, 0, cs)
        for v in range(NVEC):
            plsc.store_scatter(table, [dv[v]], zeros16)

        pltpu.async_copy(cnt, cnt_hbm.at[row], sem_c)

    def pair_body(k, carry):
        row_a = base_row + 2 * k
        process(k, row_a, ids_sA, ids_dA, sem_isA, sem_idA, cntA, sem_cA)
        process(k, row_a + 1, ids_sB, ids_dB, sem_isB, sem_idB, cntB, sem_cB)
        return carry
    lax.fori_loop(0, n_pairs, pair_body, 0)

    last_a = base_row + rows_per - 2
    pltpu.make_async_copy(cntA, cnt_hbm.at[last_a], sem_cA).wait()
    pltpu.make_async_copy(cntB, cnt_hbm.at[last_a + 1], sem_cB).wait()


def _mlp_tc_kernel(ct_ref, w1_ref, b1_ref, w2_ref, b2_ref, os_ref, od_ref):
    ctf = ct_ref[...].astype(jnp.float32)
    w1 = w1_ref[...]
    b1 = b1_ref[...]
    w2 = w2_ref[...]
    b22 = 2.0 * b2_ref[...]
    for qa, qb, oref in ((0, 1, os_ref), (2, 3, od_ref)):
        for l in range(L):
            ca = ctf[qa, l:l + 1, :]
            cb = ctf[qb, l:l + 1, :]
            h = (jnp.maximum(w1 * ca + b1, 0.0)
                 + jnp.maximum(w1 * cb + b1, 0.0))
            y = jax.lax.dot_general(w2, h, (((1,), (0,)), ((), ())),
                                    preferred_element_type=jnp.float32)
            oref[l] = y + b22


def _mlp_tc_kernel2(ct_ref, w1_ref, b1_ref, w2_ref, b2_ref, ps_ref, pd_ref,
                    os_ref, od_ref):
    _mlp_tc_kernel(ct_ref, w1_ref, b1_ref, w2_ref, b2_ref, os_ref, od_ref)


def _mlp_tc(ct3_h, W1, b1, W2, b2, half, prev=None):
    nblk = (B // 2) // BBLK
    base = half * nblk
    in_specs = [
        pl.BlockSpec((4, LPAD, BBLK), lambda i: (0, 0, i)),
        pl.BlockSpec((F, 1), lambda i: (0, 0)),
        pl.BlockSpec((F, 1), lambda i: (0, 0)),
        pl.BlockSpec((F, F), lambda i: (0, 0)),
        pl.BlockSpec((F, 1), lambda i: (0, 0)),
    ]
    args = [ct3_h, W1.reshape(F, 1), b1.reshape(F, 1), W2, b2.reshape(F, 1)]
    kwargs = {}
    body = _mlp_tc_kernel
    if prev is not None:
        in_specs += [pl.BlockSpec(memory_space=pl.ANY),
                     pl.BlockSpec(memory_space=pl.ANY)]
        args += list(prev)
        kwargs["input_output_aliases"] = {5: 0, 6: 1}
        body = _mlp_tc_kernel2
    return pl.pallas_call(
        body,
        grid=(nblk,),
        in_specs=in_specs,
        out_specs=[
            pl.BlockSpec((L, F, BBLK), lambda i: (0, 0, i + base)),
            pl.BlockSpec((L, F, BBLK), lambda i: (0, 0, i + base)),
        ],
        out_shape=[jax.ShapeDtypeStruct((L, F, B), jnp.float32),
                   jax.ShapeDtypeStruct((L, F, B), jnp.float32)],
        **kwargs,
    )(*args)


@jax.jit
def kernel(src_ids, dst_ids, W1, b1, W2, b2):
    pad = jnp.full((B, LPAD - L), PAD_ID, jnp.int32)
    src_p = jnp.concatenate([src_ids, pad], axis=1)
    dst_p = jnp.concatenate([dst_ids, pad], axis=1)

    info = plsc.get_sparse_core_info()
    nw = info.num_cores * info.num_subcores
    half = B // 2
    rows_per = half // nw

    mesh = plsc.VectorSubcoreMesh(core_axis_name="c", subcore_axis_name="s")

    def run_sc(row0):
        sck = functools.partial(
            pl.kernel,
            mesh=mesh,
            compiler_params=pltpu.CompilerParams(needs_layout_passes=False),
            out_type=jax.ShapeDtypeStruct((half, CW), jnp.int32),
            scratch_types=(
                [pltpu.VMEM((TABLE_W,), jnp.int32)]
                + [pltpu.VMEM((LPAD,), jnp.int32) for _ in range(4)]
                + [pltpu.VMEM((CW,), jnp.int32) for _ in range(2)]
                + [pltpu.SemaphoreType.DMA for _ in range(6)]
            ),
        )(functools.partial(_sc_kernel, rows_per, row0))
        return sck(src_p, dst_p)

    counts0 = run_sc(0)
    counts1 = run_sc(half)

    ct3_0 = jnp.transpose(counts0.reshape(half, 4, LPAD), (1, 2, 0))
    ct3_1 = jnp.transpose(counts1.reshape(half, 4, LPAD), (1, 2, 0))
    prev = _mlp_tc(ct3_0, W1, b1, W2, b2, 0)
    os_lfb, od_lfb = _mlp_tc(ct3_1, W1, b1, W2, b2, 1, prev=prev)
    return (jnp.transpose(os_lfb, (2, 0, 1)),
            jnp.transpose(od_lfb, (2, 0, 1)))


# speedup vs baseline: 1.0602x; 1.0602x over previous
import functools

import jax
import jax.numpy as jnp
from jax import lax
from jax.experimental import pallas as pl
from jax.experimental.pallas import tpu as pltpu
from jax.experimental.pallas import tpu_sc as plsc

B = 1024
L = 200
F = 16
V = 100000

LANES = 16
LPAD = 208
NVEC = LPAD // LANES
PAD_ID = V
TABLE_W = 100096
CW = 4 * LPAD
BBLK = 256


def _sc_kernel(rows_per, src_hbm, dst_hbm, cnt_hbm,
               table, ids_sA, ids_dA, ids_sB, ids_dB, cntA, cntB,
               sem_isA, sem_idA, sem_isB, sem_idB, sem_cA, sem_cB):
    wid = lax.axis_index("s") * 2 + lax.axis_index("c")
    base_row = wid * rows_per
    n_pairs = rows_per // 2

    zeros16 = jnp.zeros((LANES,), jnp.int32)
    ones16 = jnp.ones((LANES,), jnp.int32)

    def zinit(i, carry):
        for u in range(8):
            table[pl.ds((i * 8 + u) * LANES, LANES)] = zeros16
        return carry
    lax.fori_loop(0, TABLE_W // (8 * LANES), zinit, 0)

    pltpu.async_copy(src_hbm.at[base_row], ids_sA, sem_isA)
    pltpu.async_copy(dst_hbm.at[base_row], ids_dA, sem_idA)
    pltpu.async_copy(src_hbm.at[base_row + 1], ids_sB, sem_isB)
    pltpu.async_copy(dst_hbm.at[base_row + 1], ids_dB, sem_idB)

    def process(k, row, ids_s, ids_d, sem_is, sem_id, cnt, sem_c):
        pltpu.make_async_copy(src_hbm.at[row], ids_s, sem_is).wait()
        pltpu.make_async_copy(dst_hbm.at[row], ids_d, sem_id).wait()

        sv = [ids_s[pl.ds(v * LANES, LANES)] for v in range(NVEC)]
        dv = [ids_d[pl.ds(v * LANES, LANES)] for v in range(NVEC)]

        @pl.when(k + 1 < n_pairs)
        def _():
            pltpu.async_copy(src_hbm.at[row + 2], ids_s, sem_is)
            pltpu.async_copy(dst_hbm.at[row + 2], ids_d, sem_id)

        @pl.when(k > 0)
        def _():
            pltpu.make_async_copy(cnt, cnt_hbm.at[row - 2], sem_c).wait()

        for v in range(NVEC):
            plsc.addupdate_scatter(table, [sv[v]], ones16)
        for v in range(NVEC):
            cs = plsc.load_gather(table, [sv[v]])
            cd = plsc.load_gather(table, [dv[v]])
            cnt[pl.ds(v * LANES, LANES)] = jnp.where(sv[v] == 0, 0, cs)
            cnt[pl.ds(3 * LPAD + v * LANES, LANES)] = jnp.where(dv[v] == 0, 0, cd)
        for v in range(NVEC):
            plsc.store_scatter(table, [sv[v]], zeros16)

        for v in range(NVEC):
            plsc.addupdate_scatter(table, [dv[v]], ones16)
        for v in range(NVEC):
            cd = plsc.load_gather(table, [dv[v]])
            cs = plsc.load_gather(table, [sv[v]])
            cnt[pl.ds(2 * LPAD + v * LANES, LANES)] = jnp.where(dv[v] == 0, 0, cd)
            cnt[pl.ds(LPAD + v * LANES, LANES)] = jnp.where(sv[v] == 0, 0, cs)
        for v in range(NVEC):
            plsc.store_scatter(table, [dv[v]], zeros16)

        pltpu.async_copy(cnt, cnt_hbm.at[row], sem_c)

    def pair_body(k, carry):
        row_a = base_row + 2 * k
        process(k, row_a, ids_sA, ids_dA, sem_isA, sem_idA, cntA, sem_cA)
        process(k, row_a + 1, ids_sB, ids_dB, sem_isB, sem_idB, cntB, sem_cB)
        return carry
    lax.fori_loop(0, n_pairs, pair_body, 0)

    last_a = base_row + rows_per - 2
    pltpu.make_async_copy(cntA, cnt_hbm.at[last_a], sem_cA).wait()
    pltpu.make_async_copy(cntB, cnt_hbm.at[last_a + 1], sem_cB).wait()


def _mlp_tc_kernel(ct_ref, w1_ref, b1_ref, w2_ref, b2_ref, os_ref, od_ref):
    ctf = ct_ref[...].astype(jnp.float32)
    w1 = w1_ref[...]
    b1 = b1_ref[...]
    w2 = w2_ref[...]
    b22 = 2.0 * b2_ref[...]
    for qa, qb, oref in ((0, 1, os_ref), (2, 3, od_ref)):
        for l in range(L):
            ca = ctf[qa, l:l + 1, :]
            cb = ctf[qb, l:l + 1, :]
            h = (jnp.maximum(w1 * ca + b1, 0.0)
                 + jnp.maximum(w1 * cb + b1, 0.0))
            y = jax.lax.dot_general(w2, h, (((1,), (0,)), ((), ())),
                                    preferred_element_type=jnp.float32)
            oref[l] = y + b22


def _mlp_tc(ct3, W1, b1, W2, b2):
    nblk = B // BBLK
    return pl.pallas_call(
        _mlp_tc_kernel,
        grid=(nblk,),
        in_specs=[
            pl.BlockSpec((4, LPAD, BBLK), lambda i: (0, 0, i)),
            pl.BlockSpec((F, 1), lambda i: (0, 0)),
            pl.BlockSpec((F, 1), lambda i: (0, 0)),
            pl.BlockSpec((F, F), lambda i: (0, 0)),
            pl.BlockSpec((F, 1), lambda i: (0, 0)),
        ],
        out_specs=[
            pl.BlockSpec((L, F, BBLK), lambda i: (0, 0, i)),
            pl.BlockSpec((L, F, BBLK), lambda i: (0, 0, i)),
        ],
        out_shape=[jax.ShapeDtypeStruct((L, F, B), jnp.float32),
                   jax.ShapeDtypeStruct((L, F, B), jnp.float32)],
    )(ct3, W1.reshape(F, 1), b1.reshape(F, 1), W2, b2.reshape(F, 1))


@jax.jit
def kernel(src_ids, dst_ids, W1, b1, W2, b2):
    pad = jnp.full((B, LPAD - L), PAD_ID, jnp.int32)
    src_p = jnp.concatenate([src_ids, pad], axis=1)
    dst_p = jnp.concatenate([dst_ids, pad], axis=1)

    info = plsc.get_sparse_core_info()
    nw = info.num_cores * info.num_subcores
    rows_per = B // nw

    mesh = plsc.VectorSubcoreMesh(core_axis_name="c", subcore_axis_name="s")
    sck = functools.partial(
        pl.kernel,
        mesh=mesh,
        compiler_params=pltpu.CompilerParams(needs_layout_passes=False),
        out_type=jax.ShapeDtypeStruct((B, CW), jnp.int32),
        scratch_types=(
            [pltpu.VMEM((TABLE_W,), jnp.int32)]
            + [pltpu.VMEM((LPAD,), jnp.int32) for _ in range(4)]
            + [pltpu.VMEM((CW,), jnp.int32) for _ in range(2)]
            + [pltpu.SemaphoreType.DMA for _ in range(6)]
        ),
    )(functools.partial(_sc_kernel, rows_per))
    counts = sck(src_p, dst_p)

    ct3 = jnp.transpose(counts.reshape(B, 4, LPAD), (1, 2, 0))
    os_lfb, od_lfb = _mlp_tc(ct3, W1, b1, W2, b2)
    return (jnp.transpose(os_lfb, (2, 0, 1)),
            jnp.transpose(od_lfb, (2, 0, 1)))
